# Initial kernel scaffold; baseline (speedup 1.0000x reference)
import jax
import jax.numpy as jnp
from jax.experimental import pallas as pl


def kernel(h, edge_index, W1, b1, W2, b2, gamma, beta):
    raise NotImplementedError("write your pallas kernel here")


# speedup vs baseline: 4.7487x; 4.7487x over previous
import functools

import jax
import jax.numpy as jnp
from jax import lax
from jax.experimental import pallas as pl
from jax.experimental.pallas import tpu as pltpu
from jax.experimental.pallas import tpu_sc as plsc

N = 10000
E = 320000
D = 128

NC = 2
NS = 16
K = 128

CHUNKS_PER_WORKER = 79
E_PAD = NC * NS * CHUNKS_PER_WORKER * K
ROWS_PER_SUB = 632
N_ACC = ROWS_PER_SUB * NS


def _sc_segment_sum(h, src, dst, zeros_init):
    mesh = plsc.VectorSubcoreMesh(core_axis_name="c", subcore_axis_name="s")

    @functools.partial(
        pl.kernel,
        out_type=jax.ShapeDtypeStruct((NC, N_ACC, D), jnp.float32),
        mesh=mesh,
        scratch_types=[
            pltpu.VMEM_SHARED((N_ACC, D), jnp.float32),
            pltpu.VMEM((K,), jnp.int32),
            pltpu.VMEM((K,), jnp.int32),
            pltpu.VMEM((K, D), jnp.float32),
            pltpu.SemaphoreType.DMA,
        ],
    )
    def seg_sum(h_hbm, src_hbm, dst_hbm, zero_hbm, out_hbm, acc, src_v, dst_v,
                rows_v, sem):
        c = lax.axis_index("c")
        s = lax.axis_index("s")
        wid = c * NS + s

        pltpu.sync_copy(zero_hbm, acc.at[pl.ds(s * ROWS_PER_SUB, ROWS_PER_SUB)])
        plsc.subcore_barrier()

        def body(i, carry):
            off = (wid * CHUNKS_PER_WORKER + i) * K
            pltpu.sync_copy(src_hbm.at[pl.ds(off, K)], src_v)
            pltpu.async_copy(h_hbm.at[src_v], rows_v, sem).wait()
            pltpu.sync_copy(dst_hbm.at[pl.ds(off, K)], dst_v)
            pltpu.sync_copy(rows_v, acc.at[dst_v], add=True)
            return carry

        lax.fori_loop(0, CHUNKS_PER_WORKER, body, 0)
        plsc.subcore_barrier()

        pltpu.sync_copy(acc.at[pl.ds(s * ROWS_PER_SUB, ROWS_PER_SUB)],
                        out_hbm.at[c, pl.ds(s * ROWS_PER_SUB, ROWS_PER_SUB)])

    return seg_sum(h, src, dst, zeros_init)


def _tc_dense(h, parts, W1, b1, W2, b2, gamma, beta):
    def body(h_ref, p_ref, W1_ref, b1_ref, W2_ref, b2_ref, g_ref, bt_ref,
             out_ref):
        hh = h_ref[...]
        x = hh + p_ref[0, :N, :] + p_ref[1, :N, :]
        y = jnp.maximum(
            jnp.dot(x, W1_ref[...], preferred_element_type=jnp.float32)
            + b1_ref[...], 0.0)
        z = (jnp.dot(y, W2_ref[...], preferred_element_type=jnp.float32)
             + b2_ref[...])
        mean = jnp.mean(z, axis=0, keepdims=True)
        zc = z - mean
        var = jnp.mean(zc * zc, axis=0, keepdims=True)
        zn = zc * jax.lax.rsqrt(var + 1e-5) * g_ref[...] + bt_ref[...]
        out_ref[...] = hh + jnp.maximum(zn, 0.0)

    return pl.pallas_call(
        body,
        out_shape=jax.ShapeDtypeStruct((N, D), jnp.float32),
    )(h, parts, W1, b1.reshape(1, D), W2, b2.reshape(1, D),
      gamma.reshape(1, D), beta.reshape(1, D))


def kernel(h, edge_index, W1, b1, W2, b2, gamma, beta):
    src = edge_index[0]
    dst = edge_index[1]
    pad = E_PAD - E
    src = jnp.concatenate([src, jnp.zeros((pad,), jnp.int32)])
    dst = jnp.concatenate([dst, jnp.full((pad,), N, jnp.int32)])
    zeros_init = jnp.zeros((ROWS_PER_SUB, D), jnp.float32)
    parts = _sc_segment_sum(h, src, dst, zeros_init)
    return _tc_dense(h, parts, W1, b1, W2, b2, gamma, beta)
